# Initial kernel scaffold; baseline (speedup 1.0000x reference)
import jax
import jax.numpy as jnp
from jax.experimental import pallas as pl


def kernel(x, edge_index, W0, b0, g0, be0, W1, b1, g1, be1, W2, b2):
    raise NotImplementedError("write your pallas kernel here")


# speedup vs baseline: 13.6405x; 13.6405x over previous
import functools
import math

import jax
import jax.numpy as jnp
from jax import lax
from jax.experimental import pallas as pl
from jax.experimental.pallas import tpu as pltpu
from jax.experimental.pallas import tpu_sc as plsc

N = 10000
D = 128
E = 320000
NPAD = 10240
NTRASH = NPAD - N
NSC = 2
NSUB = 16
ROWS_PER_TILE = NPAD // NSUB
CH = 128
NCH = 80
EPAD = NSC * NSUB * NCH * CH
BN_INV = 1.0 / math.sqrt(1.0 + 1e-5)

_mesh = plsc.VectorSubcoreMesh(core_axis_name="c", subcore_axis_name="s")


@functools.partial(
    pl.kernel,
    out_type=jax.ShapeDtypeStruct((NSC, NPAD), jnp.float32),
    mesh=_mesh,
    scratch_types=[
        pltpu.VMEM((CH,), jnp.int32),
        pltpu.VMEM((CH,), jnp.float32),
        pltpu.VMEM_SHARED((NPAD,), jnp.float32),
        pltpu.SemaphoreType.DMA,
    ],
)
def _deg_kernel(dst_hbm, zero_hbm, out_hbm, didx, ones, acc, sem):
    c = lax.axis_index("c")
    s = lax.axis_index("s")
    w = c * NSUB + s
    for i in range(CH // 16):
        ones[pl.ds(i * 16, 16)] = jnp.ones((16,), jnp.float32)
    pltpu.sync_copy(zero_hbm, acc.at[pl.ds(s * ROWS_PER_TILE, ROWS_PER_TILE)])
    plsc.subcore_barrier()

    def body(j, carry):
        base = (w * NCH + j) * CH
        pltpu.sync_copy(dst_hbm.at[pl.ds(base, CH)], didx)
        pltpu.sync_copy(ones, acc.at[didx], add=True)
        return carry

    lax.fori_loop(0, NCH, body, 0)
    plsc.subcore_barrier()
    pltpu.sync_copy(acc.at[pl.ds(s * ROWS_PER_TILE, ROWS_PER_TILE)],
                    out_hbm.at[c, pl.ds(s * ROWS_PER_TILE, ROWS_PER_TILE)])


@functools.partial(
    pl.kernel,
    out_type=jax.ShapeDtypeStruct((NSC, NPAD, D), jnp.float32),
    mesh=_mesh,
    scratch_types=[
        pltpu.VMEM((CH,), jnp.int32),
        pltpu.VMEM((CH,), jnp.int32),
        pltpu.VMEM((CH, D), jnp.float32),
        pltpu.VMEM_SHARED((NPAD, D), jnp.float32),
        pltpu.SemaphoreType.DMA,
    ],
)
def _prop_kernel(y_hbm, src_hbm, dst_hbm, zrow_hbm, out_hbm, sidx, didx, rows, acc, sem):
    c = lax.axis_index("c")
    s = lax.axis_index("s")
    w = c * NSUB + s
    pltpu.sync_copy(zrow_hbm, acc.at[pl.ds(s * ROWS_PER_TILE, ROWS_PER_TILE)])
    plsc.subcore_barrier()

    def body(j, carry):
        base = (w * NCH + j) * CH
        pltpu.sync_copy(src_hbm.at[pl.ds(base, CH)], sidx)
        pltpu.async_copy(y_hbm.at[sidx], rows, sem).wait()
        pltpu.sync_copy(dst_hbm.at[pl.ds(base, CH)], didx)
        pltpu.sync_copy(rows, acc.at[didx], add=True)
        return carry

    lax.fori_loop(0, NCH, body, 0)
    plsc.subcore_barrier()
    pltpu.sync_copy(acc.at[pl.ds(s * ROWS_PER_TILE, ROWS_PER_TILE)],
                    out_hbm.at[c, pl.ds(s * ROWS_PER_TILE, ROWS_PER_TILE)])


_B = 1024
_GRID = NPAD // _B


def _mm0_body(x_ref, w_ref, degp_ref, y_ref, dinv_ref):
    deg = degp_ref[0] + degp_ref[1] + 1.0
    dinv = lax.rsqrt(deg)
    y_ref[...] = dinv * jnp.dot(x_ref[...], w_ref[...],
                                preferred_element_type=jnp.float32)
    dinv_ref[...] = dinv


def _mm_mid_body(s_ref, y_ref, dinv_ref, b_ref, g_ref, be_ref, w_ref, out_ref):
    dinv = dinv_ref[...]
    pre = dinv * (s_ref[0] + s_ref[1] + y_ref[...]) + b_ref[...]
    h = jnp.maximum(g_ref[...] * (pre * BN_INV) + be_ref[...], 0.0)
    out_ref[...] = dinv * jnp.dot(h, w_ref[...],
                                  preferred_element_type=jnp.float32)


def _final_body(s_ref, y_ref, dinv_ref, b_ref, out_ref):
    out_ref[...] = (dinv_ref[...] * (s_ref[0] + s_ref[1] + y_ref[...])
                    + b_ref[...])


_mat = pl.BlockSpec((_B, D), lambda i: (i, 0))
_vec = pl.BlockSpec((_B, 1), lambda i: (i, 0))
_row = pl.BlockSpec((1, D), lambda i: (0, 0))
_wsp = pl.BlockSpec((D, D), lambda i: (0, 0))
_par = pl.BlockSpec((NSC, _B, D), lambda i: (0, i, 0))
_f32 = jnp.float32

_mm0_call = pl.pallas_call(
    _mm0_body,
    grid=(_GRID,),
    in_specs=[_mat, _wsp, pl.BlockSpec((NSC, _B, 1), lambda i: (0, i, 0))],
    out_specs=[_mat, _vec],
    out_shape=[jax.ShapeDtypeStruct((NPAD, D), _f32),
               jax.ShapeDtypeStruct((NPAD, 1), _f32)],
)

_mm_mid_call = pl.pallas_call(
    _mm_mid_body,
    grid=(_GRID,),
    in_specs=[_par, _mat, _vec, _row, _row, _row, _wsp],
    out_specs=_mat,
    out_shape=jax.ShapeDtypeStruct((NPAD, D), _f32),
)

_final_call = pl.pallas_call(
    _final_body,
    grid=(_GRID,),
    in_specs=[_par, _mat, _vec, _row],
    out_specs=_mat,
    out_shape=jax.ShapeDtypeStruct((NPAD, D), _f32),
)


def kernel(x, edge_index, W0, b0, g0, be0, W1, b1, g1, be1, W2, b2):
    src = edge_index[0]
    dst = edge_index[1]
    pad = N + jnp.arange(EPAD - E, dtype=jnp.int32) % NTRASH
    srcp = jnp.concatenate([src, pad])
    dstp = jnp.concatenate([dst, pad])
    xp = jnp.pad(x, ((0, NPAD - N), (0, 0)))
    zvec = jnp.zeros((ROWS_PER_TILE,), _f32)
    zrow = jnp.zeros((ROWS_PER_TILE, D), _f32)

    deg_p = _deg_kernel(dstp, zvec)
    y0, dinv = _mm0_call(xp, W0, deg_p[:, :, None])
    s0 = _prop_kernel(y0, srcp, dstp, zrow)
    y1 = _mm_mid_call(s0, y0, dinv, b0[None, :], g0[None, :], be0[None, :], W1)
    s1 = _prop_kernel(y1, srcp, dstp, zrow)
    y2 = _mm_mid_call(s1, y1, dinv, b1[None, :], g1[None, :], be1[None, :], W2)
    s2 = _prop_kernel(y2, srcp, dstp, zrow)
    out = _final_call(s2, y2, dinv, b2[None, :])
    return out[:N]


# speedup vs baseline: 28.0669x; 2.0576x over previous
import functools
import math

import jax
import jax.numpy as jnp
from jax import lax
from jax.experimental import pallas as pl
from jax.experimental.pallas import tpu as pltpu
from jax.experimental.pallas import tpu_sc as plsc

N = 10000
D = 128
E = 320000
NPAD = 10240
NTRASH = NPAD - N
NSC = 2
NSUB = 16
ROWS_PER_TILE = NPAD // NSUB
CH = 128
NCH = 80
EPAD = NSC * NSUB * NCH * CH
BN_INV = 1.0 / math.sqrt(1.0 + 1e-5)

_mesh = plsc.VectorSubcoreMesh(core_axis_name="c", subcore_axis_name="s")


@functools.partial(
    pl.kernel,
    out_type=jax.ShapeDtypeStruct((NSC, NPAD), jnp.float32),
    mesh=_mesh,
    scratch_types=[
        pltpu.VMEM((NCH, CH), jnp.int32),
        pltpu.VMEM((CH,), jnp.float32),
        pltpu.VMEM_SHARED((NPAD,), jnp.float32),
        pltpu.SemaphoreType.DMA,
    ],
)
def _deg_kernel(dst_hbm, zero_hbm, out_hbm, didx, ones, acc, sem):
    c = lax.axis_index("c")
    s = lax.axis_index("s")
    w = c * NSUB + s
    for i in range(CH // 16):
        ones[pl.ds(i * 16, 16)] = jnp.ones((16,), jnp.float32)
    pltpu.sync_copy(dst_hbm.at[w], didx)
    pltpu.sync_copy(zero_hbm, acc.at[pl.ds(s * ROWS_PER_TILE, ROWS_PER_TILE)])
    plsc.subcore_barrier()

    def body(j, carry):
        pltpu.sync_copy(ones, acc.at[didx.at[j]], add=True)
        return carry

    lax.fori_loop(0, NCH, body, 0)
    plsc.subcore_barrier()
    pltpu.sync_copy(acc.at[pl.ds(s * ROWS_PER_TILE, ROWS_PER_TILE)],
                    out_hbm.at[c, pl.ds(s * ROWS_PER_TILE, ROWS_PER_TILE)])


@functools.partial(
    pl.kernel,
    out_type=jax.ShapeDtypeStruct((NSC, NPAD, D), jnp.float32),
    mesh=_mesh,
    scratch_types=[
        pltpu.VMEM((CH,), jnp.int32),
        pltpu.VMEM((CH,), jnp.int32),
        pltpu.VMEM((NCH, CH), jnp.int32),
        pltpu.VMEM((CH, D), jnp.float32),
        pltpu.VMEM((CH, D), jnp.float32),
        pltpu.VMEM_SHARED((NPAD, D), jnp.float32),
        pltpu.SemaphoreType.DMA,
        pltpu.SemaphoreType.DMA,
        pltpu.SemaphoreType.DMA,
        pltpu.SemaphoreType.DMA,
    ],
)
def _prop_kernel(y_hbm, src_hbm, dst_hbm, zrow_hbm, out_hbm,
                 sidx0, sidx1, didx, rows0, rows1, acc,
                 gsem0, gsem1, isem0, isem1):
    c = lax.axis_index("c")
    s = lax.axis_index("s")
    w = c * NSUB + s
    pltpu.sync_copy(dst_hbm.at[w], didx)
    pltpu.sync_copy(zrow_hbm, acc.at[pl.ds(s * ROWS_PER_TILE, ROWS_PER_TILE)])
    plsc.subcore_barrier()

    bufs = ((sidx0, rows0, gsem0, isem0), (sidx1, rows1, gsem1, isem1))
    for b, (sidx, rows, gsem, isem) in enumerate(bufs):
        pltpu.async_copy(src_hbm.at[w, b], sidx, isem).wait()
        pltpu.async_copy(y_hbm.at[sidx], rows, gsem)

    def body(k, carry):
        for b, (sidx, rows, gsem, isem) in enumerate(bufs):
            j = 2 * k + b
            pltpu.make_async_copy(y_hbm.at[sidx], rows, gsem).wait()

            @pl.when(j + 2 < NCH)
            def _():
                pltpu.async_copy(src_hbm.at[w, j + 2], sidx, isem)

            pltpu.sync_copy(rows, acc.at[didx.at[j]], add=True)

            @pl.when(j + 2 < NCH)
            def _():
                pltpu.make_async_copy(src_hbm.at[w, j + 2], sidx, isem).wait()
                pltpu.async_copy(y_hbm.at[sidx], rows, gsem)
        return carry

    lax.fori_loop(0, NCH // 2, body, 0)
    plsc.subcore_barrier()
    pltpu.sync_copy(acc.at[pl.ds(s * ROWS_PER_TILE, ROWS_PER_TILE)],
                    out_hbm.at[c, pl.ds(s * ROWS_PER_TILE, ROWS_PER_TILE)])


_B = 1024
_GRID = NPAD // _B


def _mm0_body(x_ref, w_ref, degp_ref, y_ref, dinv_ref):
    deg = degp_ref[0] + degp_ref[1] + 1.0
    dinv = lax.rsqrt(deg)
    y_ref[...] = dinv * jnp.dot(x_ref[...], w_ref[...],
                                preferred_element_type=jnp.float32)
    dinv_ref[...] = dinv


def _mm_mid_body(s_ref, y_ref, dinv_ref, b_ref, g_ref, be_ref, w_ref, out_ref):
    dinv = dinv_ref[...]
    pre = dinv * (s_ref[0] + s_ref[1] + y_ref[...]) + b_ref[...]
    h = jnp.maximum(g_ref[...] * (pre * BN_INV) + be_ref[...], 0.0)
    out_ref[...] = dinv * jnp.dot(h, w_ref[...],
                                  preferred_element_type=jnp.float32)


def _final_body(s_ref, y_ref, dinv_ref, b_ref, out_ref):
    out_ref[...] = (dinv_ref[...] * (s_ref[0] + s_ref[1] + y_ref[...])
                    + b_ref[...])


_mat = pl.BlockSpec((_B, D), lambda i: (i, 0))
_vec = pl.BlockSpec((_B, 1), lambda i: (i, 0))
_row = pl.BlockSpec((1, D), lambda i: (0, 0))
_wsp = pl.BlockSpec((D, D), lambda i: (0, 0))
_par = pl.BlockSpec((NSC, _B, D), lambda i: (0, i, 0))
_f32 = jnp.float32

_mm0_call = pl.pallas_call(
    _mm0_body,
    grid=(_GRID,),
    in_specs=[_mat, _wsp, pl.BlockSpec((NSC, _B, 1), lambda i: (0, i, 0))],
    out_specs=[_mat, _vec],
    out_shape=[jax.ShapeDtypeStruct((NPAD, D), _f32),
               jax.ShapeDtypeStruct((NPAD, 1), _f32)],
)

_mm_mid_call = pl.pallas_call(
    _mm_mid_body,
    grid=(_GRID,),
    in_specs=[_par, _mat, _vec, _row, _row, _row, _wsp],
    out_specs=_mat,
    out_shape=jax.ShapeDtypeStruct((NPAD, D), _f32),
)

_final_call = pl.pallas_call(
    _final_body,
    grid=(_GRID,),
    in_specs=[_par, _mat, _vec, _row],
    out_specs=_mat,
    out_shape=jax.ShapeDtypeStruct((NPAD, D), _f32),
)


def kernel(x, edge_index, W0, b0, g0, be0, W1, b1, g1, be1, W2, b2):
    src = edge_index[0]
    dst = edge_index[1]
    pad = N + jnp.arange(EPAD - E, dtype=jnp.int32) % NTRASH
    srcp = jnp.concatenate([src, pad]).reshape(NSC * NSUB, NCH, CH)
    dstp = jnp.concatenate([dst, pad]).reshape(NSC * NSUB, NCH, CH)
    xp = jnp.pad(x, ((0, NPAD - N), (0, 0)))
    zvec = jnp.zeros((ROWS_PER_TILE,), _f32)
    zrow = jnp.zeros((ROWS_PER_TILE, D), _f32)

    deg_p = _deg_kernel(dstp, zvec)
    y0, dinv = _mm0_call(xp, W0, deg_p[:, :, None])
    s0 = _prop_kernel(y0, srcp, dstp, zrow)
    y1 = _mm_mid_call(s0, y0, dinv, b0[None, :], g0[None, :], be0[None, :], W1)
    s1 = _prop_kernel(y1, srcp, dstp, zrow)
    y2 = _mm_mid_call(s1, y1, dinv, b1[None, :], g1[None, :], be1[None, :], W2)
    s2 = _prop_kernel(y2, srcp, dstp, zrow)
    out = _final_call(s2, y2, dinv, b2[None, :])
    return out[:N]


# speedup vs baseline: 28.0777x; 1.0004x over previous
import functools
import math

import jax
import jax.numpy as jnp
from jax import lax
from jax.experimental import pallas as pl
from jax.experimental.pallas import tpu as pltpu
from jax.experimental.pallas import tpu_sc as plsc

N = 10000
D = 128
E = 320000
NPAD = 10240
NTRASH = NPAD - N
NSC = 2
NSUB = 16
ROWS_PER_TILE = NPAD // NSUB
CH = 128
NCH = 80
SCB = 4
NSCB = NCH // SCB
EPAD = NSC * NSUB * NCH * CH
BN_INV = 1.0 / math.sqrt(1.0 + 1e-5)

_mesh = plsc.VectorSubcoreMesh(core_axis_name="c", subcore_axis_name="s")


@functools.partial(
    pl.kernel,
    out_type=jax.ShapeDtypeStruct((NSC, NPAD), jnp.float32),
    mesh=_mesh,
    scratch_types=[
        pltpu.VMEM((NCH, CH), jnp.int32),
        pltpu.VMEM((CH,), jnp.float32),
        pltpu.VMEM_SHARED((NPAD,), jnp.float32),
        pltpu.SemaphoreType.DMA,
    ],
)
def _deg_kernel(dst_hbm, zero_hbm, out_hbm, didx, ones, acc, sem):
    c = lax.axis_index("c")
    s = lax.axis_index("s")
    w = c * NSUB + s
    for i in range(CH // 16):
        ones[pl.ds(i * 16, 16)] = jnp.ones((16,), jnp.float32)
    pltpu.sync_copy(dst_hbm.at[w], didx)
    pltpu.sync_copy(zero_hbm, acc.at[pl.ds(s * ROWS_PER_TILE, ROWS_PER_TILE)])
    plsc.subcore_barrier()

    def body(j, carry):
        pltpu.sync_copy(ones, acc.at[didx.at[j]], add=True)
        return carry

    lax.fori_loop(0, NCH, body, 0)
    plsc.subcore_barrier()
    pltpu.sync_copy(acc.at[pl.ds(s * ROWS_PER_TILE, ROWS_PER_TILE)],
                    out_hbm.at[c, pl.ds(s * ROWS_PER_TILE, ROWS_PER_TILE)])


@functools.partial(
    pl.kernel,
    out_type=jax.ShapeDtypeStruct((NSC, NPAD, D), jnp.float32),
    mesh=_mesh,
    scratch_types=[
        pltpu.VMEM((SCB, CH), jnp.int32),
        pltpu.VMEM((SCB, CH), jnp.int32),
        pltpu.VMEM((SCB, CH), jnp.int32),
        pltpu.VMEM((SCB, CH), jnp.int32),
        pltpu.VMEM((CH, D), jnp.float32),
        pltpu.VMEM((CH, D), jnp.float32),
        pltpu.VMEM_SHARED((NPAD, D), jnp.float32),
        pltpu.SemaphoreType.DMA,
        pltpu.SemaphoreType.DMA,
        pltpu.SemaphoreType.DMA,
        pltpu.SemaphoreType.DMA,
    ],
)
def _prop_kernel(y_hbm, src_hbm, dst_hbm, zrow_hbm, out_hbm,
                 sblk0, sblk1, dblk0, dblk1, rows0, rows1, acc,
                 gsem0, gsem1, xsem0, xsem1):
    c = lax.axis_index("c")
    s = lax.axis_index("s")
    w = c * NSUB + s
    blks = ((sblk0, dblk0, xsem0), (sblk1, dblk1, xsem1))
    rbufs = ((rows0, gsem0), (rows1, gsem1))

    for p, (sblk, dblk, xsem) in enumerate(blks):
        pltpu.async_copy(src_hbm.at[w, p], sblk, xsem)
        pltpu.async_copy(dst_hbm.at[w, p], dblk, xsem)
    pltpu.sync_copy(zrow_hbm, acc.at[pl.ds(s * ROWS_PER_TILE, ROWS_PER_TILE)])
    plsc.subcore_barrier()

    pltpu.make_async_copy(src_hbm.at[w, 0], sblk0, xsem0).wait()
    pltpu.make_async_copy(dst_hbm.at[w, 0], dblk0, xsem0).wait()
    pltpu.async_copy(y_hbm.at[sblk0.at[0]], rows0, gsem0)
    pltpu.async_copy(y_hbm.at[sblk0.at[1]], rows1, gsem1)

    def body(k, carry):
        for p, (sblk, dblk, xsem) in enumerate(blks):
            sb = 2 * k + p
            osblk, odblk, oxsem = blks[1 - p]
            for i in range(SCB):
                j = sb * SCB + i
                rows, gsem = rbufs[i % 2]
                pltpu.make_async_copy(y_hbm.at[sblk.at[i]], rows, gsem).wait()
                if i == 2:
                    @pl.when(sb + 1 < NSCB)
                    def _():
                        pltpu.make_async_copy(
                            src_hbm.at[w, sb + 1], osblk, oxsem).wait()
                        pltpu.make_async_copy(
                            dst_hbm.at[w, sb + 1], odblk, oxsem).wait()
                pltpu.sync_copy(rows, acc.at[dblk.at[i]], add=True)
                nidx = sblk.at[i + 2] if i < 2 else osblk.at[i - 2]

                @pl.when(j + 2 < NCH)
                def _():
                    pltpu.async_copy(y_hbm.at[nidx], rows, gsem)

            @pl.when(sb + 2 < NSCB)
            def _():
                pltpu.async_copy(src_hbm.at[w, sb + 2], sblk, xsem)
                pltpu.async_copy(dst_hbm.at[w, sb + 2], dblk, xsem)
        return carry

    lax.fori_loop(0, NSCB // 2, body, 0)
    plsc.subcore_barrier()
    pltpu.sync_copy(acc.at[pl.ds(s * ROWS_PER_TILE, ROWS_PER_TILE)],
                    out_hbm.at[c, pl.ds(s * ROWS_PER_TILE, ROWS_PER_TILE)])


_B = 1024
_GRID = NPAD // _B


def _mm0_body(x_ref, w_ref, degp_ref, y_ref, dinv_ref):
    deg = degp_ref[0] + degp_ref[1] + 1.0
    dinv = lax.rsqrt(deg)
    y_ref[...] = dinv * jnp.dot(x_ref[...], w_ref[...],
                                preferred_element_type=jnp.float32)
    dinv_ref[...] = dinv


def _mm_mid_body(s_ref, y_ref, dinv_ref, b_ref, g_ref, be_ref, w_ref, out_ref):
    dinv = dinv_ref[...]
    pre = dinv * (s_ref[0] + s_ref[1] + y_ref[...]) + b_ref[...]
    h = jnp.maximum(g_ref[...] * (pre * BN_INV) + be_ref[...], 0.0)
    out_ref[...] = dinv * jnp.dot(h, w_ref[...],
                                  preferred_element_type=jnp.float32)


def _final_body(s_ref, y_ref, dinv_ref, b_ref, out_ref):
    out_ref[...] = (dinv_ref[...] * (s_ref[0] + s_ref[1] + y_ref[...])
                    + b_ref[...])


_mat = pl.BlockSpec((_B, D), lambda i: (i, 0))
_vec = pl.BlockSpec((_B, 1), lambda i: (i, 0))
_row = pl.BlockSpec((1, D), lambda i: (0, 0))
_wsp = pl.BlockSpec((D, D), lambda i: (0, 0))
_par = pl.BlockSpec((NSC, _B, D), lambda i: (0, i, 0))
_f32 = jnp.float32

_mm0_call = pl.pallas_call(
    _mm0_body,
    grid=(_GRID,),
    in_specs=[_mat, _wsp, pl.BlockSpec((NSC, _B, 1), lambda i: (0, i, 0))],
    out_specs=[_mat, _vec],
    out_shape=[jax.ShapeDtypeStruct((NPAD, D), _f32),
               jax.ShapeDtypeStruct((NPAD, 1), _f32)],
)

_mm_mid_call = pl.pallas_call(
    _mm_mid_body,
    grid=(_GRID,),
    in_specs=[_par, _mat, _vec, _row, _row, _row, _wsp],
    out_specs=_mat,
    out_shape=jax.ShapeDtypeStruct((NPAD, D), _f32),
)

_final_call = pl.pallas_call(
    _final_body,
    grid=(_GRID,),
    in_specs=[_par, _mat, _vec, _row],
    out_specs=_mat,
    out_shape=jax.ShapeDtypeStruct((NPAD, D), _f32),
)


def kernel(x, edge_index, W0, b0, g0, be0, W1, b1, g1, be1, W2, b2):
    src = edge_index[0]
    dst = edge_index[1]
    pad = N + jnp.arange(EPAD - E, dtype=jnp.int32) % NTRASH
    srcp = jnp.concatenate([src, pad]).reshape(NSC * NSUB, NSCB, SCB, CH)
    dstp = jnp.concatenate([dst, pad]).reshape(NSC * NSUB, NSCB, SCB, CH)
    dstp3 = dstp.reshape(NSC * NSUB, NCH, CH)
    xp = jnp.pad(x, ((0, NPAD - N), (0, 0)))
    zvec = jnp.zeros((ROWS_PER_TILE,), _f32)
    zrow = jnp.zeros((ROWS_PER_TILE, D), _f32)

    deg_p = _deg_kernel(dstp3, zvec)
    y0, dinv = _mm0_call(xp, W0, deg_p[:, :, None])
    s0 = _prop_kernel(y0, srcp, dstp, zrow)
    y1 = _mm_mid_call(s0, y0, dinv, b0[None, :], g0[None, :], be0[None, :], W1)
    s1 = _prop_kernel(y1, srcp, dstp, zrow)
    y2 = _mm_mid_call(s1, y1, dinv, b1[None, :], g1[None, :], be1[None, :], W2)
    s2 = _prop_kernel(y2, srcp, dstp, zrow)
    out = _final_call(s2, y2, dinv, b2[None, :])
    return out[:N]


# speedup vs baseline: 30.5971x; 1.0897x over previous
import functools
import math

import jax
import jax.numpy as jnp
from jax import lax
from jax.experimental import pallas as pl
from jax.experimental.pallas import tpu as pltpu
from jax.experimental.pallas import tpu_sc as plsc

N = 10000
D = 128
E = 320000
NPAD = 10112
NTRASH = NPAD - N
NSC = 2
NSUB = 16
ROWS_PER_TILE = NPAD // NSUB
CH = 128
NCH = 81
NBUF = 3
EPAD = NSC * NSUB * NCH * CH
BN_INV = 1.0 / math.sqrt(1.0 + 1e-5)

_mesh = plsc.VectorSubcoreMesh(core_axis_name="c", subcore_axis_name="s")


@functools.partial(
    pl.kernel,
    out_type=[jax.ShapeDtypeStruct((NPAD,), jnp.float32),
              jax.ShapeDtypeStruct((NPAD,), jnp.float32)],
    mesh=_mesh,
    scratch_types=[
        pltpu.VMEM((CH,), jnp.int32),
        pltpu.VMEM((CH,), jnp.int32),
        pltpu.VMEM((CH,), jnp.float32),
        pltpu.VMEM_SHARED((NPAD,), jnp.float32),
        pltpu.SemaphoreType.DMA,
        pltpu.SemaphoreType.DMA,
    ],
)
def _deg_kernel(dst_hbm, zero_hbm, out0_hbm, out1_hbm, di0, di1, ones, acc, x0, x1):
    c = lax.axis_index("c")
    s = lax.axis_index("s")
    w = c * NSUB + s
    bufs = ((di0, x0), (di1, x1))
    for b, (didx, xsem) in enumerate(bufs):
        pltpu.async_copy(dst_hbm.at[pl.ds((w * NCH + b) * CH, CH)], didx, xsem)
    for i in range(CH // 16):
        ones[pl.ds(i * 16, 16)] = jnp.ones((16,), jnp.float32)
    pltpu.sync_copy(zero_hbm, acc.at[pl.ds(s * ROWS_PER_TILE, ROWS_PER_TILE)])
    plsc.subcore_barrier()

    def body(k, carry):
        for b, (didx, xsem) in enumerate(bufs):
            j = 2 * k + b
            pltpu.make_async_copy(
                dst_hbm.at[pl.ds((w * NCH + j) * CH, CH)], didx, xsem).wait()
            pltpu.sync_copy(ones, acc.at[didx], add=True)

            @pl.when(j + 2 < NCH)
            def _():
                pltpu.async_copy(
                    dst_hbm.at[pl.ds((w * NCH + j + 2) * CH, CH)], didx, xsem)
        return carry

    lax.fori_loop(0, NCH // 2, body, 0)
    pltpu.make_async_copy(
        dst_hbm.at[pl.ds((w * NCH + NCH - 1) * CH, CH)], di0, x0).wait()
    pltpu.sync_copy(ones, acc.at[di0], add=True)
    plsc.subcore_barrier()

    @pl.when(c == 0)
    def _():
        pltpu.sync_copy(acc.at[pl.ds(s * ROWS_PER_TILE, ROWS_PER_TILE)],
                        out0_hbm.at[pl.ds(s * ROWS_PER_TILE, ROWS_PER_TILE)])

    @pl.when(c == 1)
    def _():
        pltpu.sync_copy(acc.at[pl.ds(s * ROWS_PER_TILE, ROWS_PER_TILE)],
                        out1_hbm.at[pl.ds(s * ROWS_PER_TILE, ROWS_PER_TILE)])


@functools.partial(
    pl.kernel,
    out_type=[jax.ShapeDtypeStruct((NPAD, D), jnp.float32),
              jax.ShapeDtypeStruct((NPAD, D), jnp.float32)],
    mesh=_mesh,
    scratch_types=(
        [pltpu.VMEM((CH,), jnp.int32) for _ in range(NBUF)]
        + [pltpu.VMEM((CH,), jnp.int32) for _ in range(NBUF)]
        + [pltpu.VMEM((CH, D), jnp.float32) for _ in range(NBUF)]
        + [pltpu.VMEM_SHARED((NPAD, D), jnp.float32)]
        + [pltpu.SemaphoreType.DMA for _ in range(3 * NBUF)]
    ),
)
def _prop_kernel(y_hbm, src_hbm, dst_hbm, zrow_hbm, out0_hbm, out1_hbm,
                 si0, si1, si2, di0, di1, di2, r0, r1, r2, acc,
                 g0, g1, g2, ss0, ss1, ss2, ds0, ds1, ds2):
    c = lax.axis_index("c")
    s = lax.axis_index("s")
    w = c * NSUB + s
    bufs = ((si0, di0, r0, g0, ss0, ds0), (si1, di1, r1, g1, ss1, ds1),
            (si2, di2, r2, g2, ss2, ds2))

    for b, (sidx, didx, rows, gsem, ssem, dsem) in enumerate(bufs):
        pltpu.async_copy(src_hbm.at[pl.ds((w * NCH + b) * CH, CH)], sidx, ssem)
        pltpu.async_copy(dst_hbm.at[pl.ds((w * NCH + b) * CH, CH)], didx, dsem)
    pltpu.sync_copy(zrow_hbm, acc.at[pl.ds(s * ROWS_PER_TILE, ROWS_PER_TILE)])
    plsc.subcore_barrier()

    for b, (sidx, didx, rows, gsem, ssem, dsem) in enumerate(bufs):
        pltpu.make_async_copy(src_hbm.at[pl.ds((w * NCH + b) * CH, CH)], sidx, ssem).wait()
        pltpu.async_copy(y_hbm.at[sidx], rows, gsem)

    def body(k, carry):
        for b, (sidx, didx, rows, gsem, ssem, dsem) in enumerate(bufs):
            j = NBUF * k + b
            pltpu.make_async_copy(y_hbm.at[sidx], rows, gsem).wait()

            @pl.when(j + NBUF < NCH)
            def _():
                pltpu.async_copy(src_hbm.at[pl.ds((w * NCH + j + NBUF) * CH, CH)], sidx, ssem)

            pltpu.make_async_copy(dst_hbm.at[pl.ds((w * NCH + j) * CH, CH)], didx, dsem).wait()
            pltpu.sync_copy(rows, acc.at[didx], add=True)

            @pl.when(j + NBUF < NCH)
            def _():
                pltpu.async_copy(dst_hbm.at[pl.ds((w * NCH + j + NBUF) * CH, CH)], didx, dsem)
                pltpu.make_async_copy(src_hbm.at[pl.ds((w * NCH + j + NBUF) * CH, CH)], sidx, ssem).wait()
                pltpu.async_copy(y_hbm.at[sidx], rows, gsem)
        return carry

    lax.fori_loop(0, NCH // NBUF, body, 0)
    plsc.subcore_barrier()

    @pl.when(c == 0)
    def _():
        pltpu.sync_copy(acc.at[pl.ds(s * ROWS_PER_TILE, ROWS_PER_TILE)],
                        out0_hbm.at[pl.ds(s * ROWS_PER_TILE, ROWS_PER_TILE)])

    @pl.when(c == 1)
    def _():
        pltpu.sync_copy(acc.at[pl.ds(s * ROWS_PER_TILE, ROWS_PER_TILE)],
                        out1_hbm.at[pl.ds(s * ROWS_PER_TILE, ROWS_PER_TILE)])


_B = 1264
_GRID = NPAD // _B


def _mm0_body(x_ref, w_ref, dg0_ref, dg1_ref, y_ref, dinv_ref):
    deg = dg0_ref[...] + dg1_ref[...] + 1.0
    dinv = lax.rsqrt(deg)
    y_ref[...] = dinv * jnp.dot(x_ref[...], w_ref[...],
                                preferred_element_type=jnp.float32)
    dinv_ref[...] = dinv


def _mm_mid_body(s0_ref, s1_ref, y_ref, dinv_ref, b_ref, g_ref, be_ref, w_ref, out_ref):
    dinv = dinv_ref[...]
    pre = dinv * (s0_ref[...] + s1_ref[...] + y_ref[...]) + b_ref[...]
    h = jnp.maximum(g_ref[...] * (pre * BN_INV) + be_ref[...], 0.0)
    out_ref[...] = dinv * jnp.dot(h, w_ref[...],
                                  preferred_element_type=jnp.float32)


def _final_body(s0_ref, s1_ref, y_ref, dinv_ref, b_ref, out_ref):
    out_ref[...] = (dinv_ref[...] * (s0_ref[...] + s1_ref[...] + y_ref[...])
                    + b_ref[...])


_mat = pl.BlockSpec((_B, D), lambda i: (i, 0))
_vec = pl.BlockSpec((_B, 1), lambda i: (i, 0))
_row = pl.BlockSpec((1, D), lambda i: (0, 0))
_wsp = pl.BlockSpec((D, D), lambda i: (0, 0))
_f32 = jnp.float32

_mm0_call = pl.pallas_call(
    _mm0_body,
    grid=(_GRID,),
    in_specs=[_mat, _wsp, _vec, _vec],
    out_specs=[_mat, _vec],
    out_shape=[jax.ShapeDtypeStruct((NPAD, D), _f32),
               jax.ShapeDtypeStruct((NPAD, 1), _f32)],
)

_mm_mid_call = pl.pallas_call(
    _mm_mid_body,
    grid=(_GRID,),
    in_specs=[_mat, _mat, _mat, _vec, _row, _row, _row, _wsp],
    out_specs=_mat,
    out_shape=jax.ShapeDtypeStruct((NPAD, D), _f32),
)

_final_call = pl.pallas_call(
    _final_body,
    grid=(_GRID,),
    in_specs=[_mat, _mat, _mat, _vec, _row],
    out_specs=_mat,
    out_shape=jax.ShapeDtypeStruct((NPAD, D), _f32),
)


def kernel(x, edge_index, W0, b0, g0, be0, W1, b1, g1, be1, W2, b2):
    src = edge_index[0]
    dst = edge_index[1]
    npad_e = EPAD - E
    pad_src = jnp.arange(npad_e, dtype=jnp.int32) % N
    pad_dst = N + jnp.arange(npad_e, dtype=jnp.int32) % NTRASH
    srcp = jnp.concatenate([src, pad_src])
    dstp = jnp.concatenate([dst, pad_dst])
    xp = jnp.pad(x, ((0, NPAD - N), (0, 0)))
    zvec = jnp.zeros((ROWS_PER_TILE,), _f32)
    zrow = jnp.zeros((ROWS_PER_TILE, D), _f32)

    dg0, dg1 = _deg_kernel(dstp, zvec)
    y0, dinv = _mm0_call(xp, W0, dg0[:, None], dg1[:, None])
    s0a, s0b = _prop_kernel(y0, srcp, dstp, zrow)
    y1 = _mm_mid_call(s0a, s0b, y0, dinv,
                      b0[None, :], g0[None, :], be0[None, :], W1)
    s1a, s1b = _prop_kernel(y1, srcp, dstp, zrow)
    y2 = _mm_mid_call(s1a, s1b, y1, dinv,
                      b1[None, :], g1[None, :], be1[None, :], W2)
    s2a, s2b = _prop_kernel(y2, srcp, dstp, zrow)
    out = _final_call(s2a, s2b, y2, dinv, b2[None, :])
    return out[:N]
